# baseline (speedup 1.0000x reference)
import functools
import math

import jax
import jax.numpy as jnp
from jax import lax
from jax.experimental import pallas as pl
from jax.experimental.pallas import tpu as pltpu
from jax.experimental.pallas import tpu_sc as plsc

_GAMMA = 12.0
_EPS = 2.0
_EMB_DIM = 64
_EMB_RANGE = (_GAMMA + _EPS) / _EMB_DIM
_PHASE_SCALE = math.pi / _EMB_RANGE

_B = 16384
_NC = 2
_NS = 16
_NW = _NC * _NS
_N_PER_W = _B // _NW
_CHUNK = 64
_NCHUNK = _N_PER_W // _CHUNK
_NREL = 1000
_TRIG_ROWS = 1024

_SIN_C = (9.999999944748e-01, -1.666666457030e-01, 8.333310293851e-03,
          -1.984015188491e-04, 2.752939542093e-06, -2.467649262019e-08,
          1.344998941264e-10)
_COS_C = (9.999999891118e-01, -4.999998910091e-01, 4.166648921944e-02,
          -1.388780360064e-03, 2.476988355953e-05, -2.707903084514e-07,
          1.724509092029e-09)


def _horner(x2, coef):
    r = jnp.full((16,), coef[-1], jnp.float32)
    for c in coef[-2::-1]:
        r = r * x2 + c
    return r


def _rsqrt_newton(x):
    i = lax.bitcast_convert_type(x, jnp.int32)
    i = 0x5F3759DF - lax.shift_right_arithmetic(i, 1)
    y = lax.bitcast_convert_type(i, jnp.float32)
    for _ in range(2):
        y = y * (1.5 - 0.5 * x * y * y)
    return y


def _sc_score(sample, ent, rel):
    mesh = plsc.VectorSubcoreMesh(core_axis_name="c", subcore_axis_name="s")

    buf = lambda shape, dt=jnp.float32: pltpu.VMEM(shape, dt)

    @functools.partial(
        pl.kernel,
        out_type=(
            jax.ShapeDtypeStruct((_B,), jnp.float32),
            jax.ShapeDtypeStruct((_NC * _TRIG_ROWS, 128), jnp.float32),
        ),
        mesh=mesh,
        compiler_params=pltpu.CompilerParams(needs_layout_passes=False),
        scratch_types=[
            buf((64, _EMB_DIM)),
            buf((_N_PER_W, 3), jnp.int32),
            buf((2 * _N_PER_W,), jnp.int32),
            buf((_N_PER_W,), jnp.int32),
            [buf((2 * _CHUNK, 128)) for _ in range(2)],
            [buf((_CHUNK, 128)) for _ in range(2)],
            buf((_N_PER_W,)),
            pltpu.SemaphoreType.DMA,
            pltpu.SemaphoreType.DMA,
        ],
    )
    def sc_kernel(samp_hbm, ent_hbm, rel_hbm, out_hbm, trig_hbm, rel_v,
                  samp_v, htid_v, rid_v, ht_v, trig_v, out_v,
                  sem0, sem1):
        sc = lax.axis_index("c")
        tile = lax.axis_index("s")
        wid = tile * _NC + sc
        base = wid * _N_PER_W
        lane = lax.iota(jnp.int32, 16)
        col0 = jnp.zeros((16,), jnp.int32)
        col1 = col0 + 1
        col2 = col0 + 2
        sems = (sem0, sem1)

        rows_off = jnp.minimum(tile * 64, _NREL - 64)
        pltpu.sync_copy(rel_hbm.at[pl.ds(rows_off, 64)], rel_v)
        tloc_v = ht_v[0]

        def trig_row(r, _):
            for k in range(4):
                ph = rel_v[r, pl.ds(k * 16, 16)] * _PHASE_SCALE
                x2 = ph * ph
                tloc_v[r, pl.ds(k * 16, 16)] = _horner(x2, _COS_C)
                tloc_v[r, pl.ds(64 + k * 16, 16)] = ph * _horner(x2, _SIN_C)
            return _

        lax.fori_loop(0, 64, trig_row, 0)
        pltpu.sync_copy(tloc_v.at[pl.ds(0, 64)],
                        trig_hbm.at[pl.ds(sc * _TRIG_ROWS + rows_off, 64)])
        plsc.subcore_barrier()

        pltpu.sync_copy(samp_hbm.at[pl.ds(base, _N_PER_W)], samp_v)
        gpc = _CHUNK // 16
        trig_base = sc * _TRIG_ROWS

        def extract_body(j, _):
            rows = j * 16 + lane
            c = j // gpc
            g = j - c * gpc
            hslot = pl.ds(c * 2 * _CHUNK + g * 16, 16)
            tslot = pl.ds(c * 2 * _CHUNK + _CHUNK + g * 16, 16)
            htid_v[hslot] = plsc.load_gather(samp_v, [rows, col0])
            htid_v[tslot] = plsc.load_gather(samp_v, [rows, col2])
            rid_v[pl.ds(j * 16, 16)] = (
                plsc.load_gather(samp_v, [rows, col1]) + trig_base)
            return _

        lax.fori_loop(0, _N_PER_W // 16, extract_body, 0)

        def issue(c, b):
            c = jnp.minimum(c, _NCHUNK - 1)
            pltpu.async_copy(
                ent_hbm.at[htid_v.at[pl.ds(c * 2 * _CHUNK, 2 * _CHUNK)]],
                ht_v[b], sems[b])
            pltpu.async_copy(
                trig_hbm.at[rid_v.at[pl.ds(c * _CHUNK, _CHUNK)]],
                trig_v[b], sems[b])

        def drain(b):
            pltpu.make_async_copy(
                ent_hbm.at[htid_v.at[pl.ds(0, 2 * _CHUNK)]],
                ht_v[b], sems[b]).wait()
            pltpu.make_async_copy(
                trig_hbm.at[rid_v.at[pl.ds(0, _CHUNK)]],
                trig_v[b], sems[b]).wait()

        def compute(c, b):
            ht, trig = ht_v[b], trig_v[b]

            def group_body(g, _):
                def sample_body(j, vec):
                    s = g * 16 + j
                    acc = jnp.zeros((16,), jnp.float32)
                    for k in range(4):
                        re_h = ht[s, pl.ds(k * 16, 16)]
                        im_h = ht[s, pl.ds(64 + k * 16, 16)]
                        re_t = ht[_CHUNK + s, pl.ds(k * 16, 16)]
                        im_t = ht[_CHUNK + s, pl.ds(64 + k * 16, 16)]
                        re_r = trig[s, pl.ds(k * 16, 16)]
                        im_r = trig[s, pl.ds(64 + k * 16, 16)]
                        a = re_h * re_r - im_h * im_r - re_t
                        bb = re_h * im_r + im_h * re_r - im_t
                        x = a * a + bb * bb
                        x = jnp.maximum(x, 1e-12)
                        acc = acc + x * _rsqrt_newton(x)
                    total = _GAMMA - jnp.sum(acc)
                    return jnp.where(lane == j, total, vec)

                vec = lax.fori_loop(0, 16, sample_body,
                                    jnp.zeros((16,), jnp.float32),
                                    unroll=4)
                out_v[pl.ds(c * _CHUNK + g * 16, 16)] = vec
                return _

            lax.fori_loop(0, _CHUNK // 16, group_body, 0)

        issue(0, 0)

        def pair_body(p, _):
            c0 = 2 * p
            issue(c0 + 1, 1)
            drain(0)
            compute(c0, 0)
            issue(c0 + 2, 0)
            drain(1)
            compute(c0 + 1, 1)
            return _

        lax.fori_loop(0, _NCHUNK // 2, pair_body, 0)
        drain(0)
        pltpu.sync_copy(out_v, out_hbm.at[pl.ds(base, _N_PER_W)])

    return sc_kernel(sample, ent, rel)[0]


def kernel(sample, entity_embedding, relation_embedding):
    score = _sc_score(sample, entity_embedding, relation_embedding)
    return score.reshape(_B, 1)


# speedup vs baseline: 1.0526x; 1.0526x over previous
import functools
import math

import jax
import jax.numpy as jnp
from jax import lax
from jax.experimental import pallas as pl
from jax.experimental.pallas import tpu as pltpu
from jax.experimental.pallas import tpu_sc as plsc

_GAMMA = 12.0
_EPS = 2.0
_EMB_DIM = 64
_EMB_RANGE = (_GAMMA + _EPS) / _EMB_DIM
_PHASE_SCALE = math.pi / _EMB_RANGE

_B = 16384
_NC = 2
_NS = 16
_NW = _NC * _NS
_N_PER_W = _B // _NW
_CHUNK = 64
_NCHUNK = _N_PER_W // _CHUNK
_NREL = 1000
_TRIG_ROWS = 1024

_SIN_C = (9.999999944748e-01, -1.666666457030e-01, 8.333310293851e-03,
          -1.984015188491e-04, 2.752939542093e-06, -2.467649262019e-08,
          1.344998941264e-10)
_COS_C = (9.999999891118e-01, -4.999998910091e-01, 4.166648921944e-02,
          -1.388780360064e-03, 2.476988355953e-05, -2.707903084514e-07,
          1.724509092029e-09)


def _horner(x2, coef):
    r = jnp.full((16,), coef[-1], jnp.float32)
    for c in coef[-2::-1]:
        r = r * x2 + c
    return r


def _rsqrt_newton(x):
    i = lax.bitcast_convert_type(x, jnp.int32)
    i = 0x5F3759DF - lax.shift_right_arithmetic(i, 1)
    y = lax.bitcast_convert_type(i, jnp.float32)
    xh = 0.5 * x
    for _ in range(2):
        y = y * (1.5 - xh * y * y)
    return y


def _sc_score(sample, ent, rel):
    mesh = plsc.VectorSubcoreMesh(core_axis_name="c", subcore_axis_name="s")

    buf = lambda shape, dt=jnp.float32: pltpu.VMEM(shape, dt)

    @functools.partial(
        pl.kernel,
        out_type=(
            jax.ShapeDtypeStruct((_B,), jnp.float32),
            jax.ShapeDtypeStruct((_NC * _TRIG_ROWS, 128), jnp.float32),
        ),
        mesh=mesh,
        compiler_params=pltpu.CompilerParams(needs_layout_passes=False),
        scratch_types=[
            buf((64, _EMB_DIM)),
            buf((_N_PER_W, 3), jnp.int32),
            buf((2 * _N_PER_W,), jnp.int32),
            buf((_N_PER_W,), jnp.int32),
            [buf((2 * _CHUNK, 128)) for _ in range(2)],
            [buf((_CHUNK, 128)) for _ in range(2)],
            buf((_N_PER_W,)),
            pltpu.SemaphoreType.DMA,
            pltpu.SemaphoreType.DMA,
        ],
    )
    def sc_kernel(samp_hbm, ent_hbm, rel_hbm, out_hbm, trig_hbm, rel_v,
                  samp_v, htid_v, rid_v, ht_v, trig_v, out_v,
                  sem0, sem1):
        sc = lax.axis_index("c")
        tile = lax.axis_index("s")
        wid = tile * _NC + sc
        base = wid * _N_PER_W
        lane = lax.iota(jnp.int32, 16)
        col0 = jnp.zeros((16,), jnp.int32)
        col1 = col0 + 1
        col2 = col0 + 2
        sems = (sem0, sem1)

        samp_cp = pltpu.async_copy(samp_hbm.at[pl.ds(base, _N_PER_W)],
                                   samp_v, sem1)

        rows_off = jnp.minimum(tile * 64, _NREL - 64)
        pltpu.sync_copy(rel_hbm.at[pl.ds(rows_off, 64)], rel_v)
        tloc_v = ht_v[0]

        def trig_row(r, _):
            for k in range(4):
                ph = rel_v[r, pl.ds(k * 16, 16)] * _PHASE_SCALE
                x2 = ph * ph
                tloc_v[r, pl.ds(k * 16, 16)] = _horner(x2, _COS_C)
                tloc_v[r, pl.ds(64 + k * 16, 16)] = ph * _horner(x2, _SIN_C)
            return _

        lax.fori_loop(0, 64, trig_row, 0)
        pltpu.sync_copy(tloc_v.at[pl.ds(0, 64)],
                        trig_hbm.at[pl.ds(sc * _TRIG_ROWS + rows_off, 64)])

        samp_cp.wait()
        gpc = _CHUNK // 16
        trig_base = sc * _TRIG_ROWS

        def extract_body(j, _):
            rows = j * 16 + lane
            c = j // gpc
            g = j - c * gpc
            hslot = pl.ds(c * 2 * _CHUNK + g * 16, 16)
            tslot = pl.ds(c * 2 * _CHUNK + _CHUNK + g * 16, 16)
            htid_v[hslot] = plsc.load_gather(samp_v, [rows, col0])
            htid_v[tslot] = plsc.load_gather(samp_v, [rows, col2])
            rid_v[pl.ds(j * 16, 16)] = (
                plsc.load_gather(samp_v, [rows, col1]) + trig_base)
            return _

        lax.fori_loop(0, _N_PER_W // 16, extract_body, 0)
        plsc.subcore_barrier()

        def issue(c, b):
            c = jnp.minimum(c, _NCHUNK - 1)
            pltpu.async_copy(
                ent_hbm.at[htid_v.at[pl.ds(c * 2 * _CHUNK, 2 * _CHUNK)]],
                ht_v[b], sems[b])
            pltpu.async_copy(
                trig_hbm.at[rid_v.at[pl.ds(c * _CHUNK, _CHUNK)]],
                trig_v[b], sems[b])

        def drain(b):
            pltpu.make_async_copy(
                ent_hbm.at[htid_v.at[pl.ds(0, 2 * _CHUNK)]],
                ht_v[b], sems[b]).wait()
            pltpu.make_async_copy(
                trig_hbm.at[rid_v.at[pl.ds(0, _CHUNK)]],
                trig_v[b], sems[b]).wait()

        def compute(c, b):
            ht, trig = ht_v[b], trig_v[b]

            def group_body(g, _):
                def sample_body(j, vec):
                    s = g * 16 + j
                    acc = jnp.zeros((16,), jnp.float32)
                    for k in range(4):
                        re_h = ht[s, pl.ds(k * 16, 16)]
                        im_h = ht[s, pl.ds(64 + k * 16, 16)]
                        re_t = ht[_CHUNK + s, pl.ds(k * 16, 16)]
                        im_t = ht[_CHUNK + s, pl.ds(64 + k * 16, 16)]
                        re_r = trig[s, pl.ds(k * 16, 16)]
                        im_r = trig[s, pl.ds(64 + k * 16, 16)]
                        a = re_h * re_r - im_h * im_r - re_t
                        bb = re_h * im_r + im_h * re_r - im_t
                        x = a * a + bb * bb
                        x = jnp.maximum(x, 1e-12)
                        acc = acc + x * _rsqrt_newton(x)
                    total = _GAMMA - jnp.sum(acc)
                    return jnp.where(lane == j, total, vec)

                vec = lax.fori_loop(0, 16, sample_body,
                                    jnp.zeros((16,), jnp.float32),
                                    unroll=2)
                out_v[pl.ds(c * _CHUNK + g * 16, 16)] = vec
                return _

            lax.fori_loop(0, _CHUNK // 16, group_body, 0)

        issue(0, 0)

        def pair_body(p, _):
            c0 = 2 * p
            issue(c0 + 1, 1)
            drain(0)
            compute(c0, 0)
            issue(c0 + 2, 0)
            drain(1)
            compute(c0 + 1, 1)
            return _

        lax.fori_loop(0, _NCHUNK // 2, pair_body, 0)
        drain(0)
        pltpu.sync_copy(out_v, out_hbm.at[pl.ds(base, _N_PER_W)])

    return sc_kernel(sample, ent, rel)[0]


def kernel(sample, entity_embedding, relation_embedding):
    score = _sc_score(sample, entity_embedding, relation_embedding)
    return score.reshape(_B, 1)


# speedup vs baseline: 1.0544x; 1.0017x over previous
import functools
import math

import jax
import jax.numpy as jnp
from jax import lax
from jax.experimental import pallas as pl
from jax.experimental.pallas import tpu as pltpu
from jax.experimental.pallas import tpu_sc as plsc

_GAMMA = 12.0
_EPS = 2.0
_EMB_DIM = 64
_EMB_RANGE = (_GAMMA + _EPS) / _EMB_DIM
_PHASE_SCALE = math.pi / _EMB_RANGE

_B = 16384
_NC = 2
_NS = 16
_NW = _NC * _NS
_N_PER_W = _B // _NW
_CHUNK = 64
_NCHUNK = _N_PER_W // _CHUNK
_NREL = 1000
_TRIG_ROWS = 1024

_SIN_C = (9.999995999200e-01, -1.666655263541e-01, 8.332402988790e-03,
          -1.980863334289e-04, 2.699714637300e-06, -2.036224490555e-08)
_COS_C = (9.999992107855e-01, -4.999942133863e-01, 4.165977780684e-02,
          -1.385878991970e-03, 2.420294136687e-05, -2.197296381879e-07)


def _horner(x2, coef):
    r = jnp.full((16,), coef[-1], jnp.float32)
    for c in coef[-2::-1]:
        r = r * x2 + c
    return r


def _rsqrt_newton(x):
    i = lax.bitcast_convert_type(x, jnp.int32)
    i = 0x5F3759DF - lax.shift_right_arithmetic(i, 1)
    y = lax.bitcast_convert_type(i, jnp.float32)
    xh = 0.5 * x
    for _ in range(2):
        y = y * (1.5 - (xh * y) * y)
    return y


def _sc_score(sample, ent, rel):
    mesh = plsc.VectorSubcoreMesh(core_axis_name="c", subcore_axis_name="s")

    buf = lambda shape, dt=jnp.float32: pltpu.VMEM(shape, dt)

    @functools.partial(
        pl.kernel,
        out_type=(
            jax.ShapeDtypeStruct((_B,), jnp.float32),
            jax.ShapeDtypeStruct((_NC * _TRIG_ROWS, 128), jnp.float32),
        ),
        mesh=mesh,
        compiler_params=pltpu.CompilerParams(needs_layout_passes=False),
        scratch_types=[
            buf((64, _EMB_DIM)),
            buf((_N_PER_W, 3), jnp.int32),
            buf((2 * _N_PER_W,), jnp.int32),
            buf((_N_PER_W,), jnp.int32),
            [buf((2 * _CHUNK, 128)) for _ in range(2)],
            [buf((_CHUNK, 128)) for _ in range(2)],
            buf((_N_PER_W,)),
            pltpu.SemaphoreType.DMA,
            pltpu.SemaphoreType.DMA,
        ],
    )
    def sc_kernel(samp_hbm, ent_hbm, rel_hbm, out_hbm, trig_hbm, rel_v,
                  samp_v, htid_v, rid_v, ht_v, trig_v, out_v,
                  sem0, sem1):
        sc = lax.axis_index("c")
        tile = lax.axis_index("s")
        wid = tile * _NC + sc
        base = wid * _N_PER_W
        lane = lax.iota(jnp.int32, 16)
        col0 = jnp.zeros((16,), jnp.int32)
        col1 = col0 + 1
        col2 = col0 + 2
        sems = (sem0, sem1)

        samp_cp = pltpu.async_copy(samp_hbm.at[pl.ds(base, _N_PER_W)],
                                   samp_v, sem1)

        rows_off = jnp.minimum(tile * 64, _NREL - 64)
        pltpu.sync_copy(rel_hbm.at[pl.ds(rows_off, 64)], rel_v)
        tloc_v = ht_v[0]

        def trig_row(r, _):
            for k in range(4):
                ph = rel_v[r, pl.ds(k * 16, 16)] * _PHASE_SCALE
                x2 = ph * ph
                tloc_v[r, pl.ds(k * 16, 16)] = _horner(x2, _COS_C)
                tloc_v[r, pl.ds(64 + k * 16, 16)] = ph * _horner(x2, _SIN_C)
            return _

        lax.fori_loop(0, 64, trig_row, 0)
        pltpu.sync_copy(tloc_v.at[pl.ds(0, 64)],
                        trig_hbm.at[pl.ds(sc * _TRIG_ROWS + rows_off, 64)])

        samp_cp.wait()
        gpc = _CHUNK // 16
        trig_base = sc * _TRIG_ROWS

        def extract_body(j, _):
            rows = j * 16 + lane
            c = j // gpc
            g = j - c * gpc
            hslot = pl.ds(c * 2 * _CHUNK + g * 16, 16)
            tslot = pl.ds(c * 2 * _CHUNK + _CHUNK + g * 16, 16)
            htid_v[hslot] = plsc.load_gather(samp_v, [rows, col0])
            htid_v[tslot] = plsc.load_gather(samp_v, [rows, col2])
            rid_v[pl.ds(j * 16, 16)] = (
                plsc.load_gather(samp_v, [rows, col1]) + trig_base)
            return _

        lax.fori_loop(0, _N_PER_W // 16, extract_body, 0)
        plsc.subcore_barrier()

        def issue(c, b):
            c = jnp.minimum(c, _NCHUNK - 1)
            pltpu.async_copy(
                ent_hbm.at[htid_v.at[pl.ds(c * 2 * _CHUNK, 2 * _CHUNK)]],
                ht_v[b], sems[b])
            pltpu.async_copy(
                trig_hbm.at[rid_v.at[pl.ds(c * _CHUNK, _CHUNK)]],
                trig_v[b], sems[b])

        def drain(b):
            pltpu.make_async_copy(
                ent_hbm.at[htid_v.at[pl.ds(0, 2 * _CHUNK)]],
                ht_v[b], sems[b]).wait()
            pltpu.make_async_copy(
                trig_hbm.at[rid_v.at[pl.ds(0, _CHUNK)]],
                trig_v[b], sems[b]).wait()

        def compute(c, b):
            ht, trig = ht_v[b], trig_v[b]

            def group_body(g, _):
                def sample_body(j, vec):
                    s = g * 16 + j
                    acc = jnp.zeros((16,), jnp.float32)
                    for k in range(4):
                        re_h = ht[s, pl.ds(k * 16, 16)]
                        im_h = ht[s, pl.ds(64 + k * 16, 16)]
                        re_t = ht[_CHUNK + s, pl.ds(k * 16, 16)]
                        im_t = ht[_CHUNK + s, pl.ds(64 + k * 16, 16)]
                        re_r = trig[s, pl.ds(k * 16, 16)]
                        im_r = trig[s, pl.ds(64 + k * 16, 16)]
                        a = re_h * re_r - im_h * im_r - re_t
                        bb = re_h * im_r + im_h * re_r - im_t
                        x = a * a + bb * bb
                        acc = acc + x * _rsqrt_newton(x)
                    total = _GAMMA - jnp.sum(acc)
                    return jnp.where(lane == j, total, vec)

                vec = lax.fori_loop(0, 16, sample_body,
                                    jnp.zeros((16,), jnp.float32),
                                    unroll=2)
                out_v[pl.ds(c * _CHUNK + g * 16, 16)] = vec
                return _

            lax.fori_loop(0, _CHUNK // 16, group_body, 0)

        issue(0, 0)

        def pair_body(p, _):
            c0 = 2 * p
            issue(c0 + 1, 1)
            drain(0)
            compute(c0, 0)
            issue(c0 + 2, 0)
            drain(1)
            compute(c0 + 1, 1)
            return _

        lax.fori_loop(0, _NCHUNK // 2, pair_body, 0)
        drain(0)
        pltpu.sync_copy(out_v, out_hbm.at[pl.ds(base, _N_PER_W)])

    return sc_kernel(sample, ent, rel)[0]


def kernel(sample, entity_embedding, relation_embedding):
    score = _sc_score(sample, entity_embedding, relation_embedding)
    return score.reshape(_B, 1)
